# baseline (speedup 1.0000x reference)
import functools

import jax
import jax.numpy as jnp
from jax import lax
from jax.experimental import pallas as pl
from jax.experimental.pallas import tpu as pltpu
from jax.experimental.pallas import tpu_sc as plsc

N = 10000
E = 160000
CH = 64
EPAD = 163840
EROWS = EPAD // 128
NC, NS = 2, 16
NPAD = 10016
SLAB = 624
TAIL_O = N - 15 * SLAB
TAIL_Z = NPAD - 15 * SLAB
IR = 128
NR = EPAD // IR // NS
NB = 4

_MESH = plsc.VectorSubcoreMesh(core_axis_name="c", subcore_axis_name="s")


def _zero_acc(sub, zeros, acc):
    @pl.when(sub < NS - 1)
    def _():
        pltpu.sync_copy(zeros.at[pl.ds(0, SLAB)],
                        acc.at[pl.ds(sub * SLAB, SLAB)])
    @pl.when(sub == NS - 1)
    def _():
        pltpu.sync_copy(zeros, acc.at[pl.ds(15 * SLAB, TAIL_Z)])


def _copy_out(sub, acc, out, off):
    @pl.when(sub < NS - 1)
    def _():
        pltpu.sync_copy(acc.at[pl.ds(sub * SLAB, SLAB)],
                        out.at[pl.ds(off + sub * SLAB, SLAB)])
    @pl.when(sub == NS - 1)
    def _():
        pltpu.sync_copy(acc.at[pl.ds(15 * SLAB, TAIL_O)],
                        out.at[pl.ds(off + 15 * SLAB, TAIL_O)])


def _make_segsum(P, with_deg):
    n_out = P * NC * N + (N if with_deg else 0)

    @functools.partial(
        pl.kernel,
        out_type=jax.ShapeDtypeStruct((n_out, CH), jnp.float32),
        mesh=_MESH,
        compiler_params=pltpu.CompilerParams(use_tc_tiling_on_sc=False),
        scratch_types=[
            pltpu.VMEM((NR * IR,), jnp.int32),
            pltpu.VMEM((NR, IR), jnp.int32),
            pltpu.VMEM((NB, IR, CH), jnp.float32),
            pltpu.VMEM_SHARED((NPAD, CH), jnp.float32),
            pltpu.SemaphoreType.DMA,
            [pltpu.SemaphoreType.DMA] * NB,
        ],
    )
    def segsum(h_t, src1, dst2, zeros, ones, out,
               idx_s, idx_d, rows, acc, sem_g, sem_s):
        core = lax.axis_index("c")
        sub = lax.axis_index("s")
        pltpu.sync_copy(src1.at[pl.ds(sub * NR * IR, NR * IR)], idx_s)
        pltpu.sync_copy(dst2.at[pl.ds(sub * NR, NR)], idx_d)

        def shift(delta):
            def body(i, _):
                idx_s[pl.ds(i * 16, 16)] = idx_s[pl.ds(i * 16, 16)] + delta
                return 0
            lax.fori_loop(0, NR * IR // 16, body, 0)

        NG = NR

        def fire_g(grp, buf):
            pltpu.async_copy(h_t.at[idx_s.at[pl.ds(grp * IR, IR)]],
                             rows.at[buf], sem_g)

        def wait_g(buf):
            pltpu.make_async_copy(h_t.at[idx_s.at[pl.ds(0, IR)]],
                                  rows.at[buf], sem_g).wait()

        def fire_s(grp, buf):
            pltpu.async_copy(rows.at[buf], acc.at[idx_d.at[grp]],
                             sem_s[buf], add=True)

        def wait_s(buf):
            pltpu.make_async_copy(rows.at[buf], acc.at[idx_d.at[0]],
                                  sem_s[buf]).wait()

        for p in range(P):
            shift(core * (P * N) if p == 0 else N)
            _zero_acc(sub, zeros, acc)
            plsc.subcore_barrier()

            for b in range(NB - 1):
                fire_g(b, b)

            def pipe(k, _):
                for s in range(NB):
                    t = k * NB + s
                    wait_g(s)
                    fire_s(t, s)
                    @pl.when(t > 0)
                    def _():
                        wait_s((s - 1) % NB)
                    @pl.when(t + NB - 1 < NG)
                    def _():
                        fire_g(t + NB - 1, (s - 1) % NB)
                return 0

            lax.fori_loop(0, NG // NB, pipe, 0)
            wait_s(NB - 1)
            plsc.subcore_barrier()
            _copy_out(sub, acc, out, (core * P + p) * N)

        if with_deg:
            plsc.subcore_barrier()
            pltpu.sync_copy(ones, rows.at[0])
            _zero_acc(sub, zeros, acc)
            plsc.subcore_barrier()

            def deg_body(r, _):
                pltpu.sync_copy(rows.at[0], acc.at[idx_d.at[r]], add=True)
                return 0

            lax.fori_loop(0, NR, deg_body, 0)
            plsc.subcore_barrier()
            @pl.when(core == 0)
            def _():
                _copy_out(sub, acc, out, P * NC * N)

    return segsum


_segsum_w256_deg = _make_segsum(2, True)
_segsum_w512 = _make_segsum(4, False)
_segsum_w256 = _make_segsum(2, False)

_TC_R = 2000


def _pre_body(h_ref, w_ref, b_ref, out_ref):
    out_ref[...] = jnp.dot(h_ref[...], w_ref[...],
                           preferred_element_type=jnp.float32) + b_ref[...]


def _tc_pre(h, W, b):
    fin, fout = W.shape
    return pl.pallas_call(
        _pre_body,
        grid=(N // _TC_R,),
        in_specs=[
            pl.BlockSpec((_TC_R, fin), lambda i: (i, 0)),
            pl.BlockSpec((fin, fout), lambda i: (0, 0)),
            pl.BlockSpec((1, fout), lambda i: (0, 0)),
        ],
        out_specs=pl.BlockSpec((_TC_R, fout), lambda i: (i, 0)),
        out_shape=jax.ShapeDtypeStruct((N, fout), jnp.float32),
    )(h, W, b.reshape(1, fout))


def _post_body(pre_ref, agg_ref, deg_ref, wn_ref, out_ref):
    inv = 1.0 / jnp.maximum(deg_ref[...], 1.0)
    mean = agg_ref[...] * inv
    acc = pre_ref[...] + jnp.dot(mean, wn_ref[...],
                                 preferred_element_type=jnp.float32)
    out_ref[...] = jnp.maximum(acc, 0.0)


def _tc_post(pre, agg, deg, Wn):
    fin, fout = Wn.shape
    return pl.pallas_call(
        _post_body,
        grid=(N // _TC_R,),
        in_specs=[
            pl.BlockSpec((_TC_R, fout), lambda i: (i, 0)),
            pl.BlockSpec((_TC_R, fin), lambda i: (i, 0)),
            pl.BlockSpec((_TC_R, 1), lambda i: (i, 0)),
            pl.BlockSpec((fin, fout), lambda i: (0, 0)),
        ],
        out_specs=pl.BlockSpec((_TC_R, fout), lambda i: (i, 0)),
        out_shape=jax.ShapeDtypeStruct((N, fout), jnp.float32),
    )(pre, agg, deg, Wn)


def _proj_body(h_ref, w_ref, out_ref):
    out_ref[...] = jnp.dot(h_ref[...], w_ref[...],
                           preferred_element_type=jnp.float32)


def _tc_proj(h, W):
    fin, fout = W.shape
    return pl.pallas_call(
        _proj_body,
        grid=(N // _TC_R,),
        in_specs=[
            pl.BlockSpec((_TC_R, fin), lambda i: (i, 0)),
            pl.BlockSpec((fin, fout), lambda i: (0, 0)),
        ],
        out_specs=pl.BlockSpec((_TC_R, fout), lambda i: (i, 0)),
        out_shape=jax.ShapeDtypeStruct((N, fout), jnp.float32),
    )(h, W)


def _final_body(pre_ref, agg_ref, deg_ref, out_ref):
    inv = 1.0 / jnp.maximum(deg_ref[...], 1.0)
    out_ref[...] = jnp.maximum(pre_ref[...] + agg_ref[...] * inv, 0.0)


def _tc_final(pre, agg, deg):
    fout = pre.shape[1]
    return pl.pallas_call(
        _final_body,
        grid=(N // _TC_R,),
        in_specs=[
            pl.BlockSpec((_TC_R, fout), lambda i: (i, 0)),
            pl.BlockSpec((_TC_R, fout), lambda i: (i, 0)),
            pl.BlockSpec((_TC_R, 1), lambda i: (i, 0)),
        ],
        out_specs=pl.BlockSpec((_TC_R, fout), lambda i: (i, 0)),
        out_shape=jax.ShapeDtypeStruct((N, fout), jnp.float32),
    )(pre, agg, deg)


def _pad_body(ei_ref, src_ref, dst_ref):
    pad = EPAD - E
    src_ref[...] = jnp.concatenate(
        [ei_ref[0, :], jnp.zeros((pad,), jnp.int32)]).reshape(1, EPAD)
    dst_ref[...] = jnp.concatenate(
        [ei_ref[1, :], jnp.full((pad,), N, jnp.int32)]).reshape(1, EPAD)


def _tc_pad(edge_index):
    return pl.pallas_call(
        _pad_body,
        out_shape=[jax.ShapeDtypeStruct((1, EPAD), jnp.int32),
                   jax.ShapeDtypeStruct((1, EPAD), jnp.int32)],
    )(edge_index)


def _to_chunks(h, P):
    return h.reshape(N, P * NC, CH).transpose(1, 0, 2).reshape(P * NC * N, CH)


def _from_chunks(a, P):
    return a.reshape(P * NC, N, CH).transpose(1, 0, 2).reshape(N, P * NC * CH)


def kernel(x, edge_index, Ws0, Wn0, b0, Ws1, Wn1, b1, Ws2, Wn2, b2):
    src1, dst1 = _tc_pad(edge_index.astype(jnp.int32))
    src1 = src1.reshape(EPAD)
    dst2 = dst1.reshape(EPAD // IR, IR)

    zeros = jnp.zeros((TAIL_Z, CH), jnp.float32)
    ones = jnp.ones((IR, CH), jnp.float32)

    out0 = _segsum_w256_deg(_to_chunks(x, 2), src1, dst2, zeros, ones)
    pre0 = _tc_pre(x, Ws0, b0)
    agg0 = _from_chunks(out0[: 2 * NC * N], 2)
    deg = out0[2 * NC * N :, :1]
    h1 = _tc_post(pre0, agg0, deg, Wn0)

    agg1r = _segsum_w512(_to_chunks(h1, 4), src1, dst2, zeros, ones)
    pre1 = _tc_pre(h1, Ws1, b1)
    h2 = _tc_post(pre1, _from_chunks(agg1r, 4), deg, Wn1)

    hp = _tc_proj(h2, Wn2)
    agg2r = _segsum_w256(_to_chunks(hp, 2), src1, dst2, zeros, ones)
    pre2 = _tc_pre(h2, Ws2, b2)
    return _tc_final(pre2, _from_chunks(agg2r, 2), deg)


# speedup vs baseline: 1.0852x; 1.0852x over previous
import functools

import jax
import jax.numpy as jnp
from jax import lax
from jax.experimental import pallas as pl
from jax.experimental.pallas import tpu as pltpu
from jax.experimental.pallas import tpu_sc as plsc

N = 10000
E = 160000
CH = 64
EPAD = 163840
NC, NS = 2, 16
NPAD = 10016
SLAB = 624
TAIL_O = N - 15 * SLAB
TAIL_Z = NPAD - 15 * SLAB
IR = 128
NR = EPAD // IR // NS
NB = 4

_MESH = plsc.VectorSubcoreMesh(core_axis_name="c", subcore_axis_name="s")


def _zero_acc(sub, zeros, acc):
    @pl.when(sub < NS - 1)
    def _():
        pltpu.sync_copy(zeros.at[pl.ds(0, SLAB)],
                        acc.at[pl.ds(sub * SLAB, SLAB)])
    @pl.when(sub == NS - 1)
    def _():
        pltpu.sync_copy(zeros, acc.at[pl.ds(15 * SLAB, TAIL_Z)])


def _copy_out(sub, acc, out, off):
    @pl.when(sub < NS - 1)
    def _():
        pltpu.sync_copy(acc.at[pl.ds(sub * SLAB, SLAB)],
                        out.at[pl.ds(off + sub * SLAB, SLAB)])
    @pl.when(sub == NS - 1)
    def _():
        pltpu.sync_copy(acc.at[pl.ds(15 * SLAB, TAIL_O)],
                        out.at[pl.ds(off + 15 * SLAB, TAIL_O)])


def _make_segsum(P, with_deg):
    n_out = P * NC * N + (N if with_deg else 0)

    @functools.partial(
        pl.kernel,
        out_type=jax.ShapeDtypeStruct((n_out, CH), jnp.float32),
        mesh=_MESH,
        compiler_params=pltpu.CompilerParams(use_tc_tiling_on_sc=False),
        scratch_types=[
            pltpu.VMEM((NR * IR,), jnp.int32),
            pltpu.VMEM((NR, IR), jnp.int32),
            pltpu.VMEM((NB, IR, CH), jnp.float32),
            pltpu.VMEM_SHARED((NPAD, CH), jnp.float32),
            pltpu.SemaphoreType.DMA,
            [pltpu.SemaphoreType.DMA] * NB,
        ],
    )
    def segsum(h_t, src1, dst2, zeros, ones, out,
               idx_s, idx_d, rows, acc, sem_g, sem_s):
        core = lax.axis_index("c")
        sub = lax.axis_index("s")
        pltpu.sync_copy(src1.at[pl.ds(sub * NR * IR, NR * IR)], idx_s)
        pltpu.sync_copy(dst2.at[pl.ds(sub * NR, NR)], idx_d)

        def shift(delta):
            def body(i, _):
                idx_s[pl.ds(i * 16, 16)] = idx_s[pl.ds(i * 16, 16)] + delta
                return 0
            lax.fori_loop(0, NR * IR // 16, body, 0)

        NG = NR

        def fire_g(grp, buf):
            pltpu.async_copy(h_t.at[idx_s.at[pl.ds(grp * IR, IR)]],
                             rows.at[buf], sem_g)

        def wait_g(buf):
            pltpu.make_async_copy(h_t.at[idx_s.at[pl.ds(0, IR)]],
                                  rows.at[buf], sem_g).wait()

        def fire_s(grp, buf):
            pltpu.async_copy(rows.at[buf], acc.at[idx_d.at[grp]],
                             sem_s[buf], add=True)

        def wait_s(buf):
            pltpu.make_async_copy(rows.at[buf], acc.at[idx_d.at[0]],
                                  sem_s[buf]).wait()

        for p in range(P):
            shift(core * (P * N) if p == 0 else N)
            _zero_acc(sub, zeros, acc)
            plsc.subcore_barrier()

            for b in range(NB - 1):
                fire_g(b, b)

            def pipe(k, _):
                for s in range(NB):
                    t = k * NB + s
                    wait_g(s)
                    fire_s(t, s)
                    @pl.when(t > 0)
                    def _():
                        wait_s((s - 1) % NB)
                    @pl.when(t + NB - 1 < NG)
                    def _():
                        fire_g(t + NB - 1, (s - 1) % NB)
                return 0

            lax.fori_loop(0, NG // NB, pipe, 0)
            wait_s(NB - 1)
            plsc.subcore_barrier()
            _copy_out(sub, acc, out, (core * P + p) * N)

        if with_deg:
            plsc.subcore_barrier()
            pltpu.sync_copy(ones, rows.at[0])
            _zero_acc(sub, zeros, acc)
            plsc.subcore_barrier()

            def deg_body(r, _):
                pltpu.sync_copy(rows.at[0], acc.at[idx_d.at[r]], add=True)
                return 0

            lax.fori_loop(0, NR, deg_body, 0)
            plsc.subcore_barrier()
            @pl.when(core == 0)
            def _():
                _copy_out(sub, acc, out, P * NC * N)

    return segsum


_segsum_w256_deg = _make_segsum(2, True)
_segsum_w512 = _make_segsum(4, False)
_segsum_w256 = _make_segsum(2, False)

_TC_R = 2000


def _chunk_store(out_ref, h):
    for c in range(out_ref.shape[0]):
        out_ref[c] = h[:, c * CH:(c + 1) * CH]


def _unchunk(agg_ref):
    C = agg_ref.shape[0]
    return jnp.concatenate([agg_ref[c] for c in range(C)], axis=1)


def _pre0_body(x_ref, ws_ref, b_ref, t_ref, pre_ref):
    _chunk_store(t_ref, x_ref[...])
    pre_ref[...] = jnp.dot(x_ref[...], ws_ref[...],
                           preferred_element_type=jnp.float32) + b_ref[...]


def _tc_pre0(x, Ws, b):
    fin, fout = Ws.shape
    C = fin // CH
    return pl.pallas_call(
        _pre0_body,
        grid=(N // _TC_R,),
        in_specs=[
            pl.BlockSpec((_TC_R, fin), lambda i: (i, 0)),
            pl.BlockSpec((fin, fout), lambda i: (0, 0)),
            pl.BlockSpec((1, fout), lambda i: (0, 0)),
        ],
        out_specs=[pl.BlockSpec((C, _TC_R, CH), lambda i: (0, i, 0)),
                   pl.BlockSpec((_TC_R, fout), lambda i: (i, 0))],
        out_shape=[jax.ShapeDtypeStruct((C, N, CH), jnp.float32),
                   jax.ShapeDtypeStruct((N, fout), jnp.float32)],
    )(x, Ws, b.reshape(1, fout))


def _body_01(pre_ref, agg_ref, deg_ref, wn_ref, ws_ref, b_ref, t_ref,
             npre_ref):
    inv = 1.0 / jnp.maximum(deg_ref[...], 1.0)
    mean = _unchunk(agg_ref) * inv
    h = jnp.maximum(pre_ref[...] + jnp.dot(
        mean, wn_ref[...], preferred_element_type=jnp.float32), 0.0)
    _chunk_store(t_ref, h)
    npre_ref[...] = jnp.dot(h, ws_ref[...],
                            preferred_element_type=jnp.float32) + b_ref[...]


def _tc_fuse01(pre, agg, deg, Wn, Ws, b):
    fin, fout = Wn.shape
    fo2 = Ws.shape[1]
    Ci, Co = fin // CH, fout // CH
    return pl.pallas_call(
        _body_01,
        grid=(N // _TC_R,),
        in_specs=[
            pl.BlockSpec((_TC_R, fout), lambda i: (i, 0)),
            pl.BlockSpec((Ci, _TC_R, CH), lambda i: (0, i, 0)),
            pl.BlockSpec((_TC_R, 1), lambda i: (i, 0)),
            pl.BlockSpec((fin, fout), lambda i: (0, 0)),
            pl.BlockSpec((fout, fo2), lambda i: (0, 0)),
            pl.BlockSpec((1, fo2), lambda i: (0, 0)),
        ],
        out_specs=[pl.BlockSpec((Co, _TC_R, CH), lambda i: (0, i, 0)),
                   pl.BlockSpec((_TC_R, fo2), lambda i: (i, 0))],
        out_shape=[jax.ShapeDtypeStruct((Co, N, CH), jnp.float32),
                   jax.ShapeDtypeStruct((N, fo2), jnp.float32)],
    )(pre, agg, deg, Wn, Ws, b.reshape(1, fo2))


def _body_12(pre_ref, agg_ref, deg_ref, wn_ref, wp_ref, ws_ref, b_ref,
             t_ref, npre_ref):
    inv = 1.0 / jnp.maximum(deg_ref[...], 1.0)
    mean = _unchunk(agg_ref) * inv
    h = jnp.maximum(pre_ref[...] + jnp.dot(
        mean, wn_ref[...], preferred_element_type=jnp.float32), 0.0)
    _chunk_store(t_ref, jnp.dot(h, wp_ref[...],
                                preferred_element_type=jnp.float32))
    npre_ref[...] = jnp.dot(h, ws_ref[...],
                            preferred_element_type=jnp.float32) + b_ref[...]


def _tc_fuse12(pre, agg, deg, Wn, Wp, Ws, b):
    fin, fout = Wn.shape
    fo2 = Wp.shape[1]
    Ci, Cp = fin // CH, fo2 // CH
    return pl.pallas_call(
        _body_12,
        grid=(N // _TC_R,),
        in_specs=[
            pl.BlockSpec((_TC_R, fout), lambda i: (i, 0)),
            pl.BlockSpec((Ci, _TC_R, CH), lambda i: (0, i, 0)),
            pl.BlockSpec((_TC_R, 1), lambda i: (i, 0)),
            pl.BlockSpec((fin, fout), lambda i: (0, 0)),
            pl.BlockSpec((fout, fo2), lambda i: (0, 0)),
            pl.BlockSpec((fout, fo2), lambda i: (0, 0)),
            pl.BlockSpec((1, fo2), lambda i: (0, 0)),
        ],
        out_specs=[pl.BlockSpec((Cp, _TC_R, CH), lambda i: (0, i, 0)),
                   pl.BlockSpec((_TC_R, fo2), lambda i: (i, 0))],
        out_shape=[jax.ShapeDtypeStruct((Cp, N, CH), jnp.float32),
                   jax.ShapeDtypeStruct((N, fo2), jnp.float32)],
    )(pre, agg, deg, Wn, Wp, Ws, b.reshape(1, fo2))


def _final_body(pre_ref, agg_ref, deg_ref, out_ref):
    inv = 1.0 / jnp.maximum(deg_ref[...], 1.0)
    out_ref[...] = jnp.maximum(pre_ref[...] + _unchunk(agg_ref) * inv, 0.0)


def _tc_final(pre, agg, deg):
    fout = pre.shape[1]
    C = fout // CH
    return pl.pallas_call(
        _final_body,
        grid=(N // _TC_R,),
        in_specs=[
            pl.BlockSpec((_TC_R, fout), lambda i: (i, 0)),
            pl.BlockSpec((C, _TC_R, CH), lambda i: (0, i, 0)),
            pl.BlockSpec((_TC_R, 1), lambda i: (i, 0)),
        ],
        out_specs=pl.BlockSpec((_TC_R, fout), lambda i: (i, 0)),
        out_shape=jax.ShapeDtypeStruct((N, fout), jnp.float32),
    )(pre, agg, deg)


def _pad_body(ei_ref, src_ref, dst_ref):
    pad = EPAD - E
    src_ref[...] = jnp.concatenate(
        [ei_ref[0, :], jnp.zeros((pad,), jnp.int32)]).reshape(1, EPAD)
    dst_ref[...] = jnp.concatenate(
        [ei_ref[1, :], jnp.full((pad,), N, jnp.int32)]).reshape(1, EPAD)


def _tc_pad(edge_index):
    return pl.pallas_call(
        _pad_body,
        out_shape=[jax.ShapeDtypeStruct((1, EPAD), jnp.int32),
                   jax.ShapeDtypeStruct((1, EPAD), jnp.int32)],
    )(edge_index)


def kernel(x, edge_index, Ws0, Wn0, b0, Ws1, Wn1, b1, Ws2, Wn2, b2):
    src1, dst1 = _tc_pad(edge_index.astype(jnp.int32))
    src1 = src1.reshape(EPAD)
    dst2 = dst1.reshape(EPAD // IR, IR)

    zeros = jnp.zeros((TAIL_Z, CH), jnp.float32)
    ones = jnp.ones((IR, CH), jnp.float32)

    t0, pre0 = _tc_pre0(x, Ws0, b0)
    out0 = _segsum_w256_deg(t0.reshape(2 * NC * N, CH), src1, dst2, zeros,
                            ones)
    deg = out0[2 * NC * N:, :1]
    agg0 = out0[:2 * NC * N].reshape(2 * NC, N, CH)
    t1, pre1 = _tc_fuse01(pre0, agg0, deg, Wn0, Ws1, b1)

    agg1 = _segsum_w512(t1.reshape(4 * NC * N, CH), src1, dst2, zeros, ones)
    tp, pre2 = _tc_fuse12(pre1, agg1.reshape(4 * NC, N, CH), deg, Wn1, Wn2,
                          Ws2, b2)

    agg2 = _segsum_w256(tp.reshape(2 * NC * N, CH), src1, dst2, zeros, ones)
    return _tc_final(pre2, agg2.reshape(2 * NC, N, CH), deg)


# speedup vs baseline: 1.0954x; 1.0094x over previous
import functools

import jax
import jax.numpy as jnp
from jax import lax
from jax.experimental import pallas as pl
from jax.experimental.pallas import tpu as pltpu
from jax.experimental.pallas import tpu_sc as plsc

N = 10000
E = 160000
CH = 64
EPAD = 163840
NC, NS = 2, 16
NPAD = 10016
SLAB = 624
TAIL_O = N - 15 * SLAB
TAIL_Z = NPAD - 15 * SLAB
IR = 128
NR = EPAD // IR // NS
NB = 5

_MESH = plsc.VectorSubcoreMesh(core_axis_name="c", subcore_axis_name="s")


def _zero_acc(sub, zeros, acc):
    @pl.when(sub < NS - 1)
    def _():
        pltpu.sync_copy(zeros.at[pl.ds(0, SLAB)],
                        acc.at[pl.ds(sub * SLAB, SLAB)])
    @pl.when(sub == NS - 1)
    def _():
        pltpu.sync_copy(zeros, acc.at[pl.ds(15 * SLAB, TAIL_Z)])


def _copy_out(sub, acc, out, off):
    @pl.when(sub < NS - 1)
    def _():
        pltpu.sync_copy(acc.at[pl.ds(sub * SLAB, SLAB)],
                        out.at[pl.ds(off + sub * SLAB, SLAB)])
    @pl.when(sub == NS - 1)
    def _():
        pltpu.sync_copy(acc.at[pl.ds(15 * SLAB, TAIL_O)],
                        out.at[pl.ds(off + 15 * SLAB, TAIL_O)])


def _make_segsum(P, with_deg):
    n_out = P * NC * N + (N if with_deg else 0)

    @functools.partial(
        pl.kernel,
        out_type=jax.ShapeDtypeStruct((n_out, CH), jnp.float32),
        mesh=_MESH,
        compiler_params=pltpu.CompilerParams(use_tc_tiling_on_sc=False),
        scratch_types=[
            pltpu.VMEM((NR * IR,), jnp.int32),
            pltpu.VMEM((NR, IR), jnp.int32),
            pltpu.VMEM((NB, IR, CH), jnp.float32),
            pltpu.VMEM_SHARED((NPAD, CH), jnp.float32),
            pltpu.SemaphoreType.DMA,
            [pltpu.SemaphoreType.DMA] * NB,
        ],
    )
    def segsum(h_t, src1, dst2, zeros, ones, out,
               idx_s, idx_d, rows, acc, sem_g, sem_s):
        core = lax.axis_index("c")
        sub = lax.axis_index("s")
        pltpu.sync_copy(src1.at[pl.ds(sub * NR * IR, NR * IR)], idx_s)
        pltpu.sync_copy(dst2.at[pl.ds(sub * NR, NR)], idx_d)

        def shift(delta):
            def body(i, _):
                idx_s[pl.ds(i * 16, 16)] = idx_s[pl.ds(i * 16, 16)] + delta
                return 0
            lax.fori_loop(0, NR * IR // 16, body, 0)

        NG = NR

        def fire_g(grp, buf):
            pltpu.async_copy(h_t.at[idx_s.at[pl.ds(grp * IR, IR)]],
                             rows.at[buf], sem_g)

        def wait_g(buf):
            pltpu.make_async_copy(h_t.at[idx_s.at[pl.ds(0, IR)]],
                                  rows.at[buf], sem_g).wait()

        def fire_s(grp, buf):
            pltpu.async_copy(rows.at[buf], acc.at[idx_d.at[grp]],
                             sem_s[buf], add=True)

        def wait_s(buf):
            pltpu.make_async_copy(rows.at[buf], acc.at[idx_d.at[0]],
                                  sem_s[buf]).wait()

        for p in range(P):
            shift(core * (P * N) if p == 0 else N)
            _zero_acc(sub, zeros, acc)
            plsc.subcore_barrier()

            for b in range(NB - 1):
                fire_g(b, b)

            def pipe(k, _):
                for s in range(NB):
                    t = k * NB + s
                    wait_g(s)
                    fire_s(t, s)
                    @pl.when(t > 0)
                    def _():
                        wait_s((s - 1) % NB)
                    @pl.when(t + NB - 1 < NG)
                    def _():
                        fire_g(t + NB - 1, (s - 1) % NB)
                return 0

            lax.fori_loop(0, NG // NB, pipe, 0)
            wait_s(NB - 1)
            plsc.subcore_barrier()
            _copy_out(sub, acc, out, (core * P + p) * N)

        if with_deg:
            plsc.subcore_barrier()
            pltpu.sync_copy(ones, rows.at[0])
            _zero_acc(sub, zeros, acc)
            plsc.subcore_barrier()

            def deg_body(r, _):
                pltpu.sync_copy(rows.at[0], acc.at[idx_d.at[r]], add=True)
                return 0

            lax.fori_loop(0, NR, deg_body, 0)
            plsc.subcore_barrier()
            @pl.when(core == 0)
            def _():
                _copy_out(sub, acc, out, P * NC * N)

    return segsum


_segsum_w256_deg = _make_segsum(2, True)
_segsum_w512 = _make_segsum(4, False)
_segsum_w256 = _make_segsum(2, False)

_TC_R = 2000


def _chunk_store(out_ref, h):
    for c in range(out_ref.shape[0]):
        out_ref[c] = h[:, c * CH:(c + 1) * CH]


def _unchunk(agg_ref):
    C = agg_ref.shape[0]
    return jnp.concatenate([agg_ref[c] for c in range(C)], axis=1)


def _pre0_body(x_ref, ws_ref, b_ref, t_ref, pre_ref):
    _chunk_store(t_ref, x_ref[...])
    pre_ref[...] = jnp.dot(x_ref[...], ws_ref[...],
                           preferred_element_type=jnp.float32) + b_ref[...]


def _tc_pre0(x, Ws, b):
    fin, fout = Ws.shape
    C = fin // CH
    return pl.pallas_call(
        _pre0_body,
        grid=(N // _TC_R,),
        in_specs=[
            pl.BlockSpec((_TC_R, fin), lambda i: (i, 0)),
            pl.BlockSpec((fin, fout), lambda i: (0, 0)),
            pl.BlockSpec((1, fout), lambda i: (0, 0)),
        ],
        out_specs=[pl.BlockSpec((C, _TC_R, CH), lambda i: (0, i, 0)),
                   pl.BlockSpec((_TC_R, fout), lambda i: (i, 0))],
        out_shape=[jax.ShapeDtypeStruct((C, N, CH), jnp.float32),
                   jax.ShapeDtypeStruct((N, fout), jnp.float32)],
    )(x, Ws, b.reshape(1, fout))


def _body_01(pre_ref, agg_ref, deg_ref, wn_ref, ws_ref, b_ref, t_ref,
             npre_ref):
    inv = 1.0 / jnp.maximum(deg_ref[...], 1.0)
    mean = _unchunk(agg_ref) * inv
    h = jnp.maximum(pre_ref[...] + jnp.dot(
        mean, wn_ref[...], preferred_element_type=jnp.float32), 0.0)
    _chunk_store(t_ref, h)
    npre_ref[...] = jnp.dot(h, ws_ref[...],
                            preferred_element_type=jnp.float32) + b_ref[...]


def _tc_fuse01(pre, agg, deg, Wn, Ws, b):
    fin, fout = Wn.shape
    fo2 = Ws.shape[1]
    Ci, Co = fin // CH, fout // CH
    return pl.pallas_call(
        _body_01,
        grid=(N // _TC_R,),
        in_specs=[
            pl.BlockSpec((_TC_R, fout), lambda i: (i, 0)),
            pl.BlockSpec((Ci, _TC_R, CH), lambda i: (0, i, 0)),
            pl.BlockSpec((_TC_R, 1), lambda i: (i, 0)),
            pl.BlockSpec((fin, fout), lambda i: (0, 0)),
            pl.BlockSpec((fout, fo2), lambda i: (0, 0)),
            pl.BlockSpec((1, fo2), lambda i: (0, 0)),
        ],
        out_specs=[pl.BlockSpec((Co, _TC_R, CH), lambda i: (0, i, 0)),
                   pl.BlockSpec((_TC_R, fo2), lambda i: (i, 0))],
        out_shape=[jax.ShapeDtypeStruct((Co, N, CH), jnp.float32),
                   jax.ShapeDtypeStruct((N, fo2), jnp.float32)],
    )(pre, agg, deg, Wn, Ws, b.reshape(1, fo2))


def _body_12(pre_ref, agg_ref, deg_ref, wn_ref, wp_ref, ws_ref, b_ref,
             t_ref, npre_ref):
    inv = 1.0 / jnp.maximum(deg_ref[...], 1.0)
    mean = _unchunk(agg_ref) * inv
    h = jnp.maximum(pre_ref[...] + jnp.dot(
        mean, wn_ref[...], preferred_element_type=jnp.float32), 0.0)
    _chunk_store(t_ref, jnp.dot(h, wp_ref[...],
                                preferred_element_type=jnp.float32))
    npre_ref[...] = jnp.dot(h, ws_ref[...],
                            preferred_element_type=jnp.float32) + b_ref[...]


def _tc_fuse12(pre, agg, deg, Wn, Wp, Ws, b):
    fin, fout = Wn.shape
    fo2 = Wp.shape[1]
    Ci, Cp = fin // CH, fo2 // CH
    return pl.pallas_call(
        _body_12,
        grid=(N // _TC_R,),
        in_specs=[
            pl.BlockSpec((_TC_R, fout), lambda i: (i, 0)),
            pl.BlockSpec((Ci, _TC_R, CH), lambda i: (0, i, 0)),
            pl.BlockSpec((_TC_R, 1), lambda i: (i, 0)),
            pl.BlockSpec((fin, fout), lambda i: (0, 0)),
            pl.BlockSpec((fout, fo2), lambda i: (0, 0)),
            pl.BlockSpec((fout, fo2), lambda i: (0, 0)),
            pl.BlockSpec((1, fo2), lambda i: (0, 0)),
        ],
        out_specs=[pl.BlockSpec((Cp, _TC_R, CH), lambda i: (0, i, 0)),
                   pl.BlockSpec((_TC_R, fo2), lambda i: (i, 0))],
        out_shape=[jax.ShapeDtypeStruct((Cp, N, CH), jnp.float32),
                   jax.ShapeDtypeStruct((N, fo2), jnp.float32)],
    )(pre, agg, deg, Wn, Wp, Ws, b.reshape(1, fo2))


def _final_body(pre_ref, agg_ref, deg_ref, out_ref):
    inv = 1.0 / jnp.maximum(deg_ref[...], 1.0)
    out_ref[...] = jnp.maximum(pre_ref[...] + _unchunk(agg_ref) * inv, 0.0)


def _tc_final(pre, agg, deg):
    fout = pre.shape[1]
    C = fout // CH
    return pl.pallas_call(
        _final_body,
        grid=(N // _TC_R,),
        in_specs=[
            pl.BlockSpec((_TC_R, fout), lambda i: (i, 0)),
            pl.BlockSpec((C, _TC_R, CH), lambda i: (0, i, 0)),
            pl.BlockSpec((_TC_R, 1), lambda i: (i, 0)),
        ],
        out_specs=pl.BlockSpec((_TC_R, fout), lambda i: (i, 0)),
        out_shape=jax.ShapeDtypeStruct((N, fout), jnp.float32),
    )(pre, agg, deg)


def _pad_body(ei_ref, src_ref, dst_ref):
    pad = EPAD - E
    src_ref[...] = jnp.concatenate(
        [ei_ref[0, :], jnp.zeros((pad,), jnp.int32)]).reshape(1, EPAD)
    dst_ref[...] = jnp.concatenate(
        [ei_ref[1, :], jnp.full((pad,), N, jnp.int32)]).reshape(1, EPAD)


def _tc_pad(edge_index):
    return pl.pallas_call(
        _pad_body,
        out_shape=[jax.ShapeDtypeStruct((1, EPAD), jnp.int32),
                   jax.ShapeDtypeStruct((1, EPAD), jnp.int32)],
    )(edge_index)


def kernel(x, edge_index, Ws0, Wn0, b0, Ws1, Wn1, b1, Ws2, Wn2, b2):
    src1, dst1 = _tc_pad(edge_index.astype(jnp.int32))
    src1 = src1.reshape(EPAD)
    dst2 = dst1.reshape(EPAD // IR, IR)

    zeros = jnp.zeros((TAIL_Z, CH), jnp.float32)
    ones = jnp.ones((IR, CH), jnp.float32)

    t0, pre0 = _tc_pre0(x, Ws0, b0)
    out0 = _segsum_w256_deg(t0.reshape(2 * NC * N, CH), src1, dst2, zeros,
                            ones)
    deg = out0[2 * NC * N:, :1]
    agg0 = out0[:2 * NC * N].reshape(2 * NC, N, CH)
    t1, pre1 = _tc_fuse01(pre0, agg0, deg, Wn0, Ws1, b1)

    agg1 = _segsum_w512(t1.reshape(4 * NC * N, CH), src1, dst2, zeros, ones)
    tp, pre2 = _tc_fuse12(pre1, agg1.reshape(4 * NC, N, CH), deg, Wn1, Wn2,
                          Ws2, b2)

    agg2 = _segsum_w256(tp.reshape(2 * NC * N, CH), src1, dst2, zeros, ones)
    return _tc_final(pre2, agg2.reshape(2 * NC, N, CH), deg)


# speedup vs baseline: 1.7601x; 1.6069x over previous
import functools

import jax
import jax.numpy as jnp
from jax import lax
from jax.experimental import pallas as pl
from jax.experimental.pallas import tpu as pltpu
from jax.experimental.pallas import tpu_sc as plsc

N = 10000
E = 160000
CH = 128
EPAD = 163840
NC, NS = 2, 16
NPAD = 10016
SLAB = 624
TAIL_O = N - 15 * SLAB
TAIL_Z = NPAD - 15 * SLAB
IR = 128
NR = EPAD // IR // NS
NB = 5

_MESH = plsc.VectorSubcoreMesh(core_axis_name="c", subcore_axis_name="s")


def _zero_acc(sub, zeros, acc):
    @pl.when(sub < NS - 1)
    def _():
        pltpu.sync_copy(zeros.at[pl.ds(0, SLAB)],
                        acc.at[pl.ds(sub * SLAB, SLAB)])
    @pl.when(sub == NS - 1)
    def _():
        pltpu.sync_copy(zeros, acc.at[pl.ds(15 * SLAB, TAIL_Z)])


def _copy_out(sub, acc, out, off):
    @pl.when(sub < NS - 1)
    def _():
        pltpu.sync_copy(acc.at[pl.ds(sub * SLAB, SLAB)],
                        out.at[pl.ds(off + sub * SLAB, SLAB)])
    @pl.when(sub == NS - 1)
    def _():
        pltpu.sync_copy(acc.at[pl.ds(15 * SLAB, TAIL_O)],
                        out.at[pl.ds(off + 15 * SLAB, TAIL_O)])


def _make_segsum(P, with_deg):
    n_out = P * NC * N + (N if with_deg else 0)

    @functools.partial(
        pl.kernel,
        out_type=jax.ShapeDtypeStruct((n_out, CH), jnp.bfloat16),
        mesh=_MESH,
        compiler_params=pltpu.CompilerParams(use_tc_tiling_on_sc=False),
        scratch_types=[
            pltpu.VMEM((NR * IR,), jnp.int32),
            pltpu.VMEM((NR, IR), jnp.int32),
            pltpu.VMEM((NB, IR, CH), jnp.bfloat16),
            pltpu.VMEM_SHARED((NPAD, CH), jnp.bfloat16),
            pltpu.SemaphoreType.DMA,
            [pltpu.SemaphoreType.DMA] * NB,
        ],
    )
    def segsum(h_t, src1, dst2, zeros, ones, out,
               idx_s, idx_d, rows, acc, sem_g, sem_s):
        core = lax.axis_index("c")
        sub = lax.axis_index("s")
        pltpu.sync_copy(src1.at[pl.ds(sub * NR * IR, NR * IR)], idx_s)
        pltpu.sync_copy(dst2.at[pl.ds(sub * NR, NR)], idx_d)

        def shift(delta):
            def body(i, _):
                idx_s[pl.ds(i * 16, 16)] = idx_s[pl.ds(i * 16, 16)] + delta
                return 0
            lax.fori_loop(0, NR * IR // 16, body, 0)

        NG = NR

        def fire_g(grp, buf):
            pltpu.async_copy(h_t.at[idx_s.at[pl.ds(grp * IR, IR)]],
                             rows.at[buf], sem_g)

        def wait_g(buf):
            pltpu.make_async_copy(h_t.at[idx_s.at[pl.ds(0, IR)]],
                                  rows.at[buf], sem_g).wait()

        def fire_s(grp, buf):
            pltpu.async_copy(rows.at[buf], acc.at[idx_d.at[grp]],
                             sem_s[buf], add=True)

        def wait_s(buf):
            pltpu.make_async_copy(rows.at[buf], acc.at[idx_d.at[0]],
                                  sem_s[buf]).wait()

        for p in range(P):
            shift(core * (P * N) if p == 0 else N)
            _zero_acc(sub, zeros, acc)
            plsc.subcore_barrier()

            for b in range(NB - 1):
                fire_g(b, b)

            def pipe(k, _):
                for s in range(NB):
                    t = k * NB + s
                    wait_g(s)
                    fire_s(t, s)
                    @pl.when(t > 0)
                    def _():
                        wait_s((s - 1) % NB)
                    @pl.when(t + NB - 1 < NG)
                    def _():
                        fire_g(t + NB - 1, (s - 1) % NB)
                return 0

            lax.fori_loop(0, NG // NB, pipe, 0)
            wait_s(NB - 1)
            plsc.subcore_barrier()
            _copy_out(sub, acc, out, (core * P + p) * N)

        if with_deg:
            plsc.subcore_barrier()
            pltpu.sync_copy(ones, rows.at[0])
            _zero_acc(sub, zeros, acc)
            plsc.subcore_barrier()

            def deg_body(r, _):
                pltpu.sync_copy(rows.at[0], acc.at[idx_d.at[r]], add=True)
                return 0

            lax.fori_loop(0, NR, deg_body, 0)
            plsc.subcore_barrier()
            @pl.when(core == 0)
            def _():
                _copy_out(sub, acc, out, P * NC * N)

    return segsum


_segsum_w256_deg = _make_segsum(1, True)
_segsum_w512 = _make_segsum(2, False)
_segsum_w256 = _make_segsum(1, False)

_TC_R = 2000


def _chunk_store(out_ref, h):
    for c in range(out_ref.shape[0]):
        out_ref[c] = h[:, c * CH:(c + 1) * CH].astype(jnp.bfloat16)


def _unchunk(agg_ref):
    C = agg_ref.shape[0]
    return jnp.concatenate(
        [agg_ref[c].astype(jnp.float32) for c in range(C)], axis=1)


def _pre0_body(x_ref, ws_ref, b_ref, t_ref, pre_ref):
    _chunk_store(t_ref, x_ref[...])
    pre_ref[...] = jnp.dot(x_ref[...], ws_ref[...],
                           preferred_element_type=jnp.float32) + b_ref[...]


def _tc_pre0(x, Ws, b):
    fin, fout = Ws.shape
    C = fin // CH
    return pl.pallas_call(
        _pre0_body,
        grid=(N // _TC_R,),
        in_specs=[
            pl.BlockSpec((_TC_R, fin), lambda i: (i, 0)),
            pl.BlockSpec((fin, fout), lambda i: (0, 0)),
            pl.BlockSpec((1, fout), lambda i: (0, 0)),
        ],
        out_specs=[pl.BlockSpec((C, _TC_R, CH), lambda i: (0, i, 0)),
                   pl.BlockSpec((_TC_R, fout), lambda i: (i, 0))],
        out_shape=[jax.ShapeDtypeStruct((C, N, CH), jnp.bfloat16),
                   jax.ShapeDtypeStruct((N, fout), jnp.float32)],
    )(x, Ws, b.reshape(1, fout))


def _body_01(pre_ref, agg_ref, deg_ref, wn_ref, ws_ref, b_ref, t_ref,
             npre_ref):
    inv = 1.0 / jnp.maximum(deg_ref[...], 1.0)
    mean = _unchunk(agg_ref) * inv
    h = jnp.maximum(pre_ref[...] + jnp.dot(
        mean, wn_ref[...], preferred_element_type=jnp.float32), 0.0)
    _chunk_store(t_ref, h)
    npre_ref[...] = jnp.dot(h, ws_ref[...],
                            preferred_element_type=jnp.float32) + b_ref[...]


def _tc_fuse01(pre, agg, deg, Wn, Ws, b):
    fin, fout = Wn.shape
    fo2 = Ws.shape[1]
    Ci, Co = fin // CH, fout // CH
    return pl.pallas_call(
        _body_01,
        grid=(N // _TC_R,),
        in_specs=[
            pl.BlockSpec((_TC_R, fout), lambda i: (i, 0)),
            pl.BlockSpec((Ci, _TC_R, CH), lambda i: (0, i, 0)),
            pl.BlockSpec((_TC_R, 1), lambda i: (i, 0)),
            pl.BlockSpec((fin, fout), lambda i: (0, 0)),
            pl.BlockSpec((fout, fo2), lambda i: (0, 0)),
            pl.BlockSpec((1, fo2), lambda i: (0, 0)),
        ],
        out_specs=[pl.BlockSpec((Co, _TC_R, CH), lambda i: (0, i, 0)),
                   pl.BlockSpec((_TC_R, fo2), lambda i: (i, 0))],
        out_shape=[jax.ShapeDtypeStruct((Co, N, CH), jnp.bfloat16),
                   jax.ShapeDtypeStruct((N, fo2), jnp.float32)],
    )(pre, agg, deg, Wn, Ws, b.reshape(1, fo2))


def _body_12(pre_ref, agg_ref, deg_ref, wn_ref, wp_ref, ws_ref, b_ref,
             t_ref, npre_ref):
    inv = 1.0 / jnp.maximum(deg_ref[...], 1.0)
    mean = _unchunk(agg_ref) * inv
    h = jnp.maximum(pre_ref[...] + jnp.dot(
        mean, wn_ref[...], preferred_element_type=jnp.float32), 0.0)
    _chunk_store(t_ref, jnp.dot(h, wp_ref[...],
                                preferred_element_type=jnp.float32))
    npre_ref[...] = jnp.dot(h, ws_ref[...],
                            preferred_element_type=jnp.float32) + b_ref[...]


def _tc_fuse12(pre, agg, deg, Wn, Wp, Ws, b):
    fin, fout = Wn.shape
    fo2 = Wp.shape[1]
    Ci, Cp = fin // CH, fo2 // CH
    return pl.pallas_call(
        _body_12,
        grid=(N // _TC_R,),
        in_specs=[
            pl.BlockSpec((_TC_R, fout), lambda i: (i, 0)),
            pl.BlockSpec((Ci, _TC_R, CH), lambda i: (0, i, 0)),
            pl.BlockSpec((_TC_R, 1), lambda i: (i, 0)),
            pl.BlockSpec((fin, fout), lambda i: (0, 0)),
            pl.BlockSpec((fout, fo2), lambda i: (0, 0)),
            pl.BlockSpec((fout, fo2), lambda i: (0, 0)),
            pl.BlockSpec((1, fo2), lambda i: (0, 0)),
        ],
        out_specs=[pl.BlockSpec((Cp, _TC_R, CH), lambda i: (0, i, 0)),
                   pl.BlockSpec((_TC_R, fo2), lambda i: (i, 0))],
        out_shape=[jax.ShapeDtypeStruct((Cp, N, CH), jnp.bfloat16),
                   jax.ShapeDtypeStruct((N, fo2), jnp.float32)],
    )(pre, agg, deg, Wn, Wp, Ws, b.reshape(1, fo2))


def _final_body(pre_ref, agg_ref, deg_ref, out_ref):
    inv = 1.0 / jnp.maximum(deg_ref[...], 1.0)
    out_ref[...] = jnp.maximum(pre_ref[...] + _unchunk(agg_ref) * inv, 0.0)


def _tc_final(pre, agg, deg):
    fout = pre.shape[1]
    C = fout // CH
    return pl.pallas_call(
        _final_body,
        grid=(N // _TC_R,),
        in_specs=[
            pl.BlockSpec((_TC_R, fout), lambda i: (i, 0)),
            pl.BlockSpec((C, _TC_R, CH), lambda i: (0, i, 0)),
            pl.BlockSpec((_TC_R, 1), lambda i: (i, 0)),
        ],
        out_specs=pl.BlockSpec((_TC_R, fout), lambda i: (i, 0)),
        out_shape=jax.ShapeDtypeStruct((N, fout), jnp.float32),
    )(pre, agg, deg)


def _pad_body(ei_ref, src_ref, dst_ref):
    pad = EPAD - E
    src_ref[...] = jnp.concatenate(
        [ei_ref[0, :], jnp.zeros((pad,), jnp.int32)]).reshape(1, EPAD)
    dst_ref[...] = jnp.concatenate(
        [ei_ref[1, :], jnp.full((pad,), N, jnp.int32)]).reshape(1, EPAD)


def _tc_pad(edge_index):
    return pl.pallas_call(
        _pad_body,
        out_shape=[jax.ShapeDtypeStruct((1, EPAD), jnp.int32),
                   jax.ShapeDtypeStruct((1, EPAD), jnp.int32)],
    )(edge_index)


def kernel(x, edge_index, Ws0, Wn0, b0, Ws1, Wn1, b1, Ws2, Wn2, b2):
    src1, dst1 = _tc_pad(edge_index.astype(jnp.int32))
    src1 = src1.reshape(EPAD)
    dst2 = dst1.reshape(EPAD // IR, IR)

    zeros = jnp.zeros((TAIL_Z, CH), jnp.bfloat16)
    ones = jnp.ones((IR, CH), jnp.bfloat16)

    t0, pre0 = _tc_pre0(x, Ws0, b0)
    out0 = _segsum_w256_deg(t0.reshape(NC * N, CH), src1, dst2, zeros, ones)
    deg = out0[NC * N:, :1].astype(jnp.float32)
    agg0 = out0[:NC * N].reshape(NC, N, CH)
    t1, pre1 = _tc_fuse01(pre0, agg0, deg, Wn0, Ws1, b1)

    agg1 = _segsum_w512(t1.reshape(2 * NC * N, CH), src1, dst2, zeros, ones)
    tp, pre2 = _tc_fuse12(pre1, agg1.reshape(2 * NC, N, CH), deg, Wn1, Wn2,
                          Ws2, b2)

    agg2 = _segsum_w256(tp.reshape(NC * N, CH), src1, dst2, zeros, ones)
    return _tc_final(pre2, agg2.reshape(NC, N, CH), deg)
